# baseline (speedup 1.0000x reference)
import jax
import jax.numpy as jnp
from jax.experimental import pallas as pl

_NUM_PROPOSALS = 1000


def _sigmoid_body(s_ref, o_ref):
    o_ref[...] = jax.nn.sigmoid(s_ref[...])


def kernel(fused_feat, W1, b1, W2, b2):
    B, C, H, W = fused_feat.shape
    N = H * W
    K = min(_NUM_PROPOSALS, N)
    dn = ('NCHW', 'OIHW', 'NCHW')
    h = jax.lax.conv_general_dilated(fused_feat, W1, window_strides=(1, 1),
                                     padding='SAME', dimension_numbers=dn)
    h = jax.nn.relu(h + b1[None, :, None, None])
    s = jax.lax.conv_general_dilated(h, W2, window_strides=(1, 1),
                                     padding='SAME', dimension_numbers=dn)
    s = s + b2[None, :, None, None]
    s_flat = s.reshape(B, N)
    scores_flat = pl.pallas_call(
        _sigmoid_body,
        out_shape=jax.ShapeDtypeStruct((B, N), jnp.float32),
    )(s_flat)
    top_scores, top_indices = jax.lax.top_k(scores_flat, K)
    ys = (top_indices // W).astype(jnp.float32) / H
    xs = (top_indices % W).astype(jnp.float32) / W
    coords = jnp.stack([xs, ys], axis=-1)
    return (top_scores, top_indices, coords)


# speedup vs baseline: 1.4917x; 1.4917x over previous
import functools

import jax
import jax.numpy as jnp
from jax.experimental import pallas as pl

_NUM_PROPOSALS = 1000


def _shift_cols(t, k):
    if k == 0:
        return t
    m = t.shape[0]
    if k > 0:
        return jnp.concatenate(
            [jnp.zeros((m, k), t.dtype), t[:, :-k]], axis=1)
    return jnp.concatenate(
        [t[:, -k:], jnp.zeros((m, -k), t.dtype)], axis=1)


def _conv_score_body(x_ref, w1_ref, b1_ref, w2_ref, b2_ref, o_ref,
                     *, H, W, CO, NB):
    N = H * W
    colmod = jax.lax.broadcasted_iota(jnp.int32, (1, NB), 1) % W
    b1 = b1_ref[...]
    w2 = w2_ref[...]
    b2 = b2_ref[0, 0]
    for j in range(N // NB):
        n0 = j * NB
        sections = []
        for dy in (-1, 0, 1):
            src = n0 + dy * W
            if src < 0:
                xs = x_ref[0, :, 0:NB]
                post = W
            elif src + NB > N:
                xs = x_ref[0, :, N - NB:N]
                post = -W
            else:
                xs = x_ref[0, :, src:src + NB]
                post = 0
            for dx in (-1, 0, 1):
                if dx == 1:
                    sec = jnp.concatenate(
                        [xs[:, 1:], jnp.zeros((xs.shape[0], 1), xs.dtype)],
                        axis=1)
                    sec = jnp.where(colmod == W - 1,
                                    jnp.bfloat16(0), sec)
                elif dx == -1:
                    sec = jnp.concatenate(
                        [jnp.zeros((xs.shape[0], 1), xs.dtype), xs[:, :-1]],
                        axis=1)
                    sec = jnp.where(colmod == 0, jnp.bfloat16(0), sec)
                else:
                    sec = xs
                sections.append(_shift_cols(sec, post))
        im2col = jnp.concatenate(sections, axis=0)
        h = jnp.dot(w1_ref[...], im2col,
                    preferred_element_type=jnp.float32)
        hr = jnp.maximum(h + b1, 0.0)
        s = jnp.dot(w2, hr.astype(jnp.bfloat16),
                    preferred_element_type=jnp.float32)
        o_ref[0, 0, n0:n0 + NB] = jax.nn.sigmoid(s[0] + b2)


def _conv_scores(fused_feat, W1, b1, W2, b2, interpret=False):
    B, C, H, Wd = fused_feat.shape
    N = H * Wd
    CO = W1.shape[0]
    NB = min(2048, N)
    x = fused_feat.reshape(B, C, N).astype(jnp.bfloat16)
    w1r = jnp.transpose(W1, (0, 2, 3, 1)).reshape(CO, 9 * C).astype(jnp.bfloat16)
    w2r = W2.reshape(1, CO).astype(jnp.bfloat16)
    b1r = b1.reshape(CO, 1)
    b2r = b2.reshape(1, 1)
    return pl.pallas_call(
        functools.partial(_conv_score_body, H=H, W=Wd, CO=CO, NB=NB),
        grid=(B,),
        in_specs=[
            pl.BlockSpec((1, C, N), lambda b: (b, 0, 0)),
            pl.BlockSpec((CO, 9 * C), lambda b: (0, 0)),
            pl.BlockSpec((CO, 1), lambda b: (0, 0)),
            pl.BlockSpec((1, CO), lambda b: (0, 0)),
            pl.BlockSpec((1, 1), lambda b: (0, 0)),
        ],
        out_specs=pl.BlockSpec((1, 1, N), lambda b: (b, 0, 0)),
        out_shape=jax.ShapeDtypeStruct((B, 1, N), jnp.float32),
        interpret=interpret,
    )(x, w1r, b1r, w2r, b2r).reshape(B, N)


def kernel(fused_feat, W1, b1, W2, b2):
    B, C, H, Wd = fused_feat.shape
    N = H * Wd
    K = min(_NUM_PROPOSALS, N)
    scores = _conv_scores(fused_feat, W1, b1, W2, b2)
    top_scores, top_indices = jax.lax.top_k(scores, K)
    ys = (top_indices // Wd).astype(jnp.float32) / H
    xs = (top_indices % Wd).astype(jnp.float32) / Wd
    coords = jnp.stack([xs, ys], axis=-1)
    return (top_scores, top_indices, coords)
